# baseline (speedup 1.0000x reference)
import functools

import jax
import jax.numpy as jnp
from jax import lax
from jax.experimental import pallas as pl
from jax.experimental.pallas import tpu as pltpu
from jax.experimental.pallas import tpu_sc as plsc

_N = 100000
_D = 128
_L = 16
_G = _D // _L
_NC, _NS = 2, 16
_NW = _NC * _NS

_N_SC = 36000
_N_TC = _N - _N_SC
_RW = _N_SC // _NW
_CH = 225
_NCHUNK = _RW // _CH

_TC_BLOCK = 4000

_mesh = plsc.VectorSubcoreMesh(core_axis_name="c", subcore_axis_name="s")


@functools.partial(
    pl.kernel,
    mesh=_mesh,
    out_type=jax.ShapeDtypeStruct((_NC * _D,), jnp.float32),
    scratch_types=[
        pltpu.VMEM((_CH * _D,), jnp.float32),
        pltpu.VMEM((_CH * _D,), jnp.float32),
        pltpu.VMEM((_D,), jnp.float32),
        pltpu.VMEM((_NS * _D,), jnp.float32),
        pltpu.VMEM_SHARED((_NS * _D,), jnp.float32),
        pltpu.SemaphoreType.DMA,
        pltpu.SemaphoreType.DMA,
    ],
)
def _sc_sum(x_hbm, out_hbm, buf0, buf1, part, gath, shared, sem0, sem1):
    c = lax.axis_index("c")
    s = lax.axis_index("s")
    wid = s * _NC + c
    base = (_N_TC + wid * _RW) * _D
    bufs, sems = (buf0, buf1), (sem0, sem1)

    def accum_rows(buf, nrows, acc):
        def row_body(r, carry):
            return tuple(carry[j] + buf[pl.ds(r * _D + j * _L, _L)]
                         for j in range(_G))
        return plsc.parallel_loop(0, nrows, 1, unroll=8, carry=acc)(row_body)

    copies = [None] * _NCHUNK
    copies[0] = pltpu.async_copy(x_hbm.at[pl.ds(base, _CH * _D)], buf0, sem0)
    acc = tuple(jnp.zeros((_L,), jnp.float32) for _ in range(_G))
    for k in range(_NCHUNK):
        if k + 1 < _NCHUNK:
            copies[k + 1] = pltpu.async_copy(
                x_hbm.at[pl.ds(base + (k + 1) * (_CH * _D), _CH * _D)],
                bufs[(k + 1) % 2], sems[(k + 1) % 2])
        copies[k].wait()
        acc = accum_rows(bufs[k % 2], _CH, acc)

    for j in range(_G):
        part[pl.ds(j * _L, _L)] = acc[j]
    pltpu.sync_copy(part, shared.at[pl.ds(s * _D, _D)])
    plsc.subcore_barrier()

    @pl.when(s == 0)
    def _combine():
        pltpu.sync_copy(shared, gath)
        tot = accum_rows(
            gath, _NS,
            tuple(jnp.zeros((_L,), jnp.float32) for _ in range(_G)))
        for j in range(_G):
            part[pl.ds(j * _L, _L)] = tot[j]
        pltpu.sync_copy(part, out_hbm.at[pl.ds(c * _D, _D)])


def _tc_body(x_ref, o_ref, acc_ref):
    i = pl.program_id(0)

    @pl.when(i == 0)
    def _init():
        acc_ref[...] = jnp.zeros_like(acc_ref)

    ones = jnp.ones((1, _TC_BLOCK), jnp.float32)
    acc_ref[...] += jnp.dot(ones, x_ref[...],
                            preferred_element_type=jnp.float32)

    @pl.when(i == pl.num_programs(0) - 1)
    def _finish():
        o_ref[...] = acc_ref[...]


def _tc_sum(x):
    grid = _N_TC // _TC_BLOCK
    return pl.pallas_call(
        _tc_body,
        grid=(grid,),
        in_specs=[pl.BlockSpec((_TC_BLOCK, _D), lambda i: (i, 0))],
        out_specs=pl.BlockSpec((1, _D), lambda i: (0, 0)),
        out_shape=jax.ShapeDtypeStruct((1, _D), jnp.float32),
        scratch_shapes=[pltpu.VMEM((1, _D), jnp.float32)],
    )(x)


def kernel(x):
    sc_partials = _sc_sum(x.reshape(-1))
    tc_partial = _tc_sum(x)
    return tc_partial + jnp.sum(sc_partials.reshape(_NC, _D),
                                axis=0, keepdims=True)


# speedup vs baseline: 2.0518x; 2.0518x over previous
import jax
import jax.numpy as jnp
from jax.experimental import pallas as pl
from jax.experimental.pallas import tpu as pltpu

_N = 100000
_D = 128
_BLOCK = 20000


def _sum_body(x_ref, o_ref, acc_ref):
    i = pl.program_id(0)

    @pl.when(i == 0)
    def _init():
        acc_ref[...] = jnp.zeros_like(acc_ref)

    ones = jnp.ones((1, _BLOCK), jnp.float32)
    acc_ref[...] += jnp.dot(ones, x_ref[...],
                            preferred_element_type=jnp.float32)

    @pl.when(i == pl.num_programs(0) - 1)
    def _finish():
        o_ref[...] = acc_ref[...]


def kernel(x):
    grid = _N // _BLOCK
    return pl.pallas_call(
        _sum_body,
        grid=(grid,),
        in_specs=[pl.BlockSpec((_BLOCK, _D), lambda i: (i, 0))],
        out_specs=pl.BlockSpec((1, _D), lambda i: (0, 0)),
        out_shape=jax.ShapeDtypeStruct((1, _D), jnp.float32),
        scratch_shapes=[pltpu.VMEM((1, _D), jnp.float32)],
    )(x)


# speedup vs baseline: 2.1667x; 1.0560x over previous
import jax
import jax.numpy as jnp
from jax.experimental import pallas as pl
from jax.experimental.pallas import tpu as pltpu

_N = 100000
_D = 128
_BLOCK = 10000


def _sum_body(x_ref, o_ref, acc_ref):
    i = pl.program_id(0)

    @pl.when(i == 0)
    def _init():
        acc_ref[...] = jnp.zeros_like(acc_ref)

    ones = jnp.ones((1, _BLOCK), jnp.float32)
    acc_ref[...] += jnp.dot(ones, x_ref[...],
                            preferred_element_type=jnp.float32)

    @pl.when(i == pl.num_programs(0) - 1)
    def _finish():
        o_ref[...] = acc_ref[...]


def kernel(x):
    grid = _N // _BLOCK
    return pl.pallas_call(
        _sum_body,
        grid=(grid,),
        in_specs=[pl.BlockSpec((_BLOCK, _D), lambda i: (i, 0))],
        out_specs=pl.BlockSpec((1, _D), lambda i: (0, 0)),
        out_shape=jax.ShapeDtypeStruct((1, _D), jnp.float32),
        scratch_shapes=[pltpu.VMEM((1, _D), jnp.float32)],
    )(x)
